# baseline (speedup 1.0000x reference)
import jax
import jax.numpy as jnp
from jax.experimental import pallas as pl


def _conv(x, p, act=True):
    y = x @ p["W"] + p["b"]
    y = y * p["g"] + p["beta"]
    return jax.nn.leaky_relu(y, 0.2) if act else y


def _gather(f, idx):
    return jnp.take(f[0], idx[0], axis=0)[None]


def _rel_pos(xyz, nidx):
    neigh = _gather(xyz, nidx)
    tile = jnp.broadcast_to(xyz[:, :, None, :], neigh.shape)
    rel = tile - neigh
    dist = jnp.sqrt(jnp.sum(rel * rel, axis=-1, keepdims=True) + 1e-12)
    return jnp.concatenate([dist, rel, tile, neigh], axis=-1)


def _att_pool(fset, att):
    sc = jax.nn.softmax(fset @ att["attW"], axis=2)
    agg = jnp.sum(fset * sc, axis=2)
    return _conv(agg, att["mlp"])


def _building_block(xyz, f_pc, nidx, p):
    f_xyz = _conv(_rel_pos(xyz, nidx), p["mlp_xyz1"])
    f_neigh = _gather(f_pc, nidx)
    f_agg = _att_pool(jnp.concatenate([f_neigh, f_xyz], axis=-1), p["att1"])
    f_xyz2 = _conv(f_xyz, p["mlp_xyz2"])
    f_neigh2 = _gather(f_agg, nidx)
    return _att_pool(jnp.concatenate([f_neigh2, f_xyz2], axis=-1), p["att2"])


def _drb(feat, xyz, nidx, p):
    f = _conv(feat, p["mlp1"])
    f = _building_block(xyz, f, nidx, p)
    f = _conv(f, p["mlp2"], act=False)
    sc = _conv(feat, p["shortcut"], act=False)
    return jax.nn.leaky_relu(f + sc, 0.2)


def _identity_pallas(x):
    def body(x_ref, o_ref):
        o_ref[...] = x_ref[...]
    return pl.pallas_call(
        body, out_shape=jax.ShapeDtypeStruct(x.shape, x.dtype))(x)


def kernel(features, xyz_0, xyz_1, xyz_2, xyz_3, neigh_idx_0, neigh_idx_1, neigh_idx_2, neigh_idx_3, sub_idx_0, sub_idx_1, sub_idx_2, sub_idx_3, interp_idx_0, interp_idx_1, interp_idx_2, interp_idx_3, params):
    xyzs = [xyz_0, xyz_1, xyz_2, xyz_3]
    nidxs = [neigh_idx_0, neigh_idx_1, neigh_idx_2, neigh_idx_3]
    sidxs = [sub_idx_0, sub_idx_1, sub_idx_2, sub_idx_3]
    iidxs = [interp_idx_0, interp_idx_1, interp_idx_2, interp_idx_3]
    x = features @ params["fc0"]["W"] + params["fc0"]["b"]
    x = jax.nn.leaky_relu(x * params["bn0"]["g"] + params["bn0"]["beta"], 0.2)
    enc_list = []
    for i in range(4):
        fe = _drb(x, xyzs[i], nidxs[i], params["enc"][i])
        fs = jnp.max(_gather(fe, sidxs[i]), axis=2)
        if i == 0:
            enc_list.append(fe)
        enc_list.append(fs)
        x = fs
    x = _conv(x, params["decoder_0"])
    for j in range(4):
        fi = _gather(x, iidxs[3 - j])[:, :, 0, :]
        x = _conv(jnp.concatenate([enc_list[-j - 2], fi], axis=-1), params["dec"][j])
    x = _conv(x, params["fc1"])
    x = _conv(x, params["fc2"])
    x = x @ params["fc"]["W"] + params["fc"]["b"]
    x = _identity_pallas(x)
    return jnp.transpose(x, (0, 2, 1))


# speedup vs baseline: 1.2190x; 1.2190x over previous
import functools

import jax
import jax.numpy as jnp
from jax import lax
from jax.experimental import pallas as pl

NS = [45056, 11264, 2816, 704, 176]
K = 16
D_OUT = [16, 64, 128, 256]
D2 = [d // 2 for d in D_OUT]
D_IN = [8, 32, 128, 256]


def _pad16(c):
    return ((c + 15) // 16) * 16


def _leaky(y):
    return jnp.where(y >= 0, y, 0.2 * y)


def _fold(p, pad_out=0):
    w = p["W"] * p["g"][None, :]
    b = p["b"] * p["g"] + p["beta"]
    if pad_out:
        w = jnp.pad(w, ((0, 0), (0, pad_out)))
        b = jnp.pad(b, (0, pad_out))
    return w, b


def _wspec():
    return pl.BlockSpec(memory_space=pl.ANY)


def _full(shape):
    nd = len(shape)
    return pl.BlockSpec(shape, lambda n, _nd=nd: (0,) * _nd)


def _blk(bn, *rest):
    shape = (bn,) + rest
    nd = len(shape)
    return pl.BlockSpec(shape, lambda n, _nd=nd: (n,) + (0,) * (_nd - 1))


def _sc_gather(table, idx):
    return jnp.take(table, idx, axis=0)


def _stage_a0(features, xyz, fc0, m1, bn):
    n = NS[0]
    d2 = D2[0]
    dg = _pad16(d2 + 3)

    def body(feat_ref, xyz_ref, fw_ref, fb_ref, mw_ref, mb_ref, t_ref, x_ref):
        x = _leaky(feat_ref[...] @ fw_ref[...] + fb_ref[...])
        f = _leaky(x @ mw_ref[...] + mb_ref[...])
        pad = jnp.zeros((bn, dg - d2 - 3), jnp.float32)
        t_ref[...] = jnp.concatenate([f, xyz_ref[...], pad], axis=-1)
        x_ref[...] = x

    fw, fb = fc0
    mw, mb = m1
    return pl.pallas_call(
        body,
        grid=(n // bn,),
        in_specs=[_blk(bn, 3), _blk(bn, 3), _full(fw.shape), _full(fb.shape),
                  _full(mw.shape), _full(mb.shape)],
        out_specs=[_blk(bn, dg), _blk(bn, 8)],
        out_shape=[jax.ShapeDtypeStruct((n, dg), jnp.float32),
                   jax.ShapeDtypeStruct((n, 8), jnp.float32)],
    )(features, xyz, fw, fb, mw, mb)


def _stage_a(i, pooled, xyz, m1, bn):
    n = NS[i]
    dfi = D_IN[i]
    d2 = D2[i]
    dg = _pad16(d2 + 3)

    def body(p_ref, xyz_ref, mw_ref, mb_ref, t_ref, x_ref):
        x = jnp.max(p_ref[...], axis=1)
        f = _leaky(x @ mw_ref[...] + mb_ref[...])
        pad = jnp.zeros((bn, dg - d2 - 3), jnp.float32)
        t_ref[...] = jnp.concatenate([f, xyz_ref[...], pad], axis=-1)
        x_ref[...] = x

    mw, mb = m1
    return pl.pallas_call(
        body,
        grid=(n // bn,),
        in_specs=[_blk(bn, K, dfi), _blk(bn, 3), _full(mw.shape), _full(mb.shape)],
        out_specs=[_blk(bn, dg), _blk(bn, dfi)],
        out_shape=[jax.ShapeDtypeStruct((n, dg), jnp.float32),
                   jax.ShapeDtypeStruct((n, dfi), jnp.float32)],
    )(pooled, xyz, mw, mb)


def _stage_a4(pooled, d0, bn):
    n = NS[4]
    c = 512

    def body(p_ref, w_ref, b_ref, o_ref):
        x = jnp.max(p_ref[...], axis=1)
        o_ref[...] = _leaky(x @ w_ref[...] + b_ref[...])

    w, b = d0
    return pl.pallas_call(
        body,
        grid=(n // bn,),
        in_specs=[_blk(bn, K, c), _full(w.shape), _full(b.shape)],
        out_specs=[_blk(bn, c)],
        out_shape=[jax.ShapeDtypeStruct((n, c), jnp.float32)],
    )(pooled, w, b)[0]


def _stage_d(i, g1, xyz3, xw1, attw, am, xw2, bn):
    n = NS[i]
    d2 = D2[i]
    dg = _pad16(d2 + 3)
    d2p = _pad16(d2)

    def body(g_ref, xyz_ref, x1w_ref, x1b_ref, aw_ref, amw_ref, amb_ref,
             x2w_ref, x2b_ref, fx2_ref, fagg_ref):
        g1b = g_ref[...]
        fnb = g1b[:, :, 0:d2]
        neigh = g1b[:, :, d2:d2 + 3]
        tile = jnp.broadcast_to(xyz_ref[...], (bn, K, 3))
        rel = tile - neigh
        dist = jnp.sqrt(jnp.sum(rel * rel, axis=-1, keepdims=True) + 1e-12)
        relf = jnp.concatenate([dist, rel, tile, neigh], axis=-1)
        relf2 = relf.reshape(bn * K, 10)
        fxyz = _leaky(relf2 @ x1w_ref[...] + x1b_ref[...])
        fset = jnp.concatenate([fnb.reshape(bn * K, d2), fxyz], axis=-1)
        logits = (fset @ aw_ref[...]).reshape(bn, K, 2 * d2)
        m = jnp.max(logits, axis=1, keepdims=True)
        e = jnp.exp(logits - m)
        sc = e / jnp.sum(e, axis=1, keepdims=True)
        agg = jnp.sum(fset.reshape(bn, K, 2 * d2) * sc, axis=1)
        fagg_ref[...] = _leaky(agg @ amw_ref[...] + amb_ref[...])
        fx2_ref[...] = _leaky(fxyz @ x2w_ref[...] + x2b_ref[...]).reshape(bn, K, d2)

    x1w, x1b = xw1
    amw, amb = am
    x2w, x2b = xw2
    return pl.pallas_call(
        body,
        grid=(n // bn,),
        in_specs=[_blk(bn, K, dg), _blk(bn, 1, 3), _full(x1w.shape),
                  _full(x1b.shape), _full(attw.shape), _full(amw.shape),
                  _full(amb.shape), _full(x2w.shape), _full(x2b.shape)],
        out_specs=[_blk(bn, K, d2), _blk(bn, d2p)],
        out_shape=[jax.ShapeDtypeStruct((n, K, d2), jnp.float32),
                   jax.ShapeDtypeStruct((n, d2p), jnp.float32)],
    )(g1, xyz3, x1w, x1b, attw, amw, amb, x2w, x2b)


def _stage_f(i, g2, fxyz2, x, attw, am2, m2, shc, bn):
    n = NS[i]
    d2 = D2[i]
    d2p = _pad16(d2)
    dout = D_OUT[i]
    dfi = D_IN[i]

    def body(g_ref, fx_ref, x_ref, aw_ref, amw_ref, amb_ref, m2w_ref,
             m2b_ref, sw_ref, sb_ref, fe_ref):
        fset = jnp.concatenate(
            [g_ref[...][:, :, 0:d2], fx_ref[...]], axis=-1).reshape(bn * K, 2 * d2)
        logits = (fset @ aw_ref[...]).reshape(bn, K, 2 * d2)
        m = jnp.max(logits, axis=1, keepdims=True)
        e = jnp.exp(logits - m)
        sc = e / jnp.sum(e, axis=1, keepdims=True)
        agg = jnp.sum(fset.reshape(bn, K, 2 * d2) * sc, axis=1)
        a = _leaky(agg @ amw_ref[...] + amb_ref[...])
        f = a @ m2w_ref[...] + m2b_ref[...]
        s = x_ref[...] @ sw_ref[...] + sb_ref[...]
        fe_ref[...] = _leaky(f + s)

    amw, amb = am2
    m2w, m2b = m2
    sw, sb = shc
    return pl.pallas_call(
        body,
        grid=(n // bn,),
        in_specs=[_blk(bn, K, d2p), _blk(bn, K, d2), _blk(bn, dfi),
                  _full(attw.shape), _full(amw.shape), _full(amb.shape),
                  _full(m2w.shape), _full(m2b.shape), _full(sw.shape),
                  _full(sb.shape)],
        out_specs=[_blk(bn, 2 * dout)],
        out_shape=[jax.ShapeDtypeStruct((n, 2 * dout), jnp.float32)],
    )(g2, fxyz2, x, attw, amw, amb, m2w, m2b, sw, sb)[0]


def _stage_dec(n, fi, skip, w_b, bn):
    cs = skip.shape[1]
    ct = fi.shape[1]
    w, b = w_b

    def body(fi_ref, s_ref, w_ref, b_ref, o_ref):
        cat = jnp.concatenate([s_ref[...], fi_ref[...]], axis=-1)
        o_ref[...] = _leaky(cat @ w_ref[...] + b_ref[...])

    return pl.pallas_call(
        body,
        grid=(n // bn,),
        in_specs=[_blk(bn, ct), _blk(bn, cs), _full(w.shape), _full(b.shape)],
        out_specs=[_blk(bn, w.shape[1])],
        out_shape=[jax.ShapeDtypeStruct((n, w.shape[1]), jnp.float32)],
    )(fi[:n], skip, w, b)[0]


def _stage_head(fi, skip, dec3, fc1, fc2, fc, bn):
    n = NS[0]

    def body(fi_ref, s_ref, dw_ref, db_ref, w1_ref, b1_ref, w2_ref, b2_ref,
             w3_ref, b3_ref, o_ref):
        cat = jnp.concatenate([s_ref[...], fi_ref[...]], axis=-1)
        x = _leaky(cat @ dw_ref[...] + db_ref[...])
        x = _leaky(x @ w1_ref[...] + b1_ref[...])
        x = _leaky(x @ w2_ref[...] + b2_ref[...])
        o_ref[...] = x @ w3_ref[...] + b3_ref[...]

    dw, db = dec3
    w1, b1 = fc1
    w2, b2 = fc2
    w3, b3 = fc
    return pl.pallas_call(
        body,
        grid=(n // bn,),
        in_specs=[_blk(bn, 32), _blk(bn, 32)] + [
            _full(a.shape) for a in (dw, db, w1, b1, w2, b2, w3, b3)],
        out_specs=[_blk(bn, 19)],
        out_shape=[jax.ShapeDtypeStruct((n, 19), jnp.float32)],
    )(fi, skip, dw, db, w1, b1, w2, b2, w3, b3)[0]


def kernel(features, xyz_0, xyz_1, xyz_2, xyz_3, neigh_idx_0, neigh_idx_1,
           neigh_idx_2, neigh_idx_3, sub_idx_0, sub_idx_1, sub_idx_2,
           sub_idx_3, interp_idx_0, interp_idx_1, interp_idx_2, interp_idx_3,
           params):
    xyzs = [xyz_0[0], xyz_1[0], xyz_2[0], xyz_3[0]]
    nidxs = [neigh_idx_0[0].reshape(-1), neigh_idx_1[0].reshape(-1),
             neigh_idx_2[0].reshape(-1), neigh_idx_3[0].reshape(-1)]
    sidxs = [sub_idx_0[0].reshape(-1), sub_idx_1[0].reshape(-1),
             sub_idx_2[0].reshape(-1), sub_idx_3[0].reshape(-1)]
    iidxs = [interp_idx_0[0].reshape(-1), interp_idx_1[0].reshape(-1),
             interp_idx_2[0].reshape(-1), interp_idx_3[0].reshape(-1)]

    p = params
    fc0w = p["fc0"]["W"] * p["bn0"]["g"][None, :]
    fc0b = p["fc0"]["b"] * p["bn0"]["g"] + p["bn0"]["beta"]

    bns = [512, 512, 704, 352]
    fe0 = None
    x = None
    skips = []
    t = None
    for i in range(4):
        ep = p["enc"][i]
        d2 = D2[i]
        if i == 0:
            t, x = _stage_a0(features[0], xyzs[0], (fc0w, fc0b),
                             _fold(ep["mlp1"]), bns[0])
        else:
            pooled = _sc_gather(fe_prev, sidxs[i - 1]).reshape(
                NS[i], K, 2 * D_OUT[i - 1])
            t, x = _stage_a(i, pooled, xyzs[i], _fold(ep["mlp1"]), bns[i])
            skips.append(x)
        g1 = _sc_gather(t, nidxs[i]).reshape(NS[i], K, _pad16(d2 + 3))
        fxyz2, fagg = _stage_d(
            i, g1, xyzs[i].reshape(NS[i], 1, 3), _fold(ep["mlp_xyz1"]),
            ep["att1"]["attW"], _fold(ep["att1"]["mlp"], _pad16(d2) - d2),
            _fold(ep["mlp_xyz2"]), bns[i])
        g2 = _sc_gather(fagg, nidxs[i]).reshape(NS[i], K, _pad16(d2))
        fe = _stage_f(i, g2, fxyz2, x, ep["att2"]["attW"],
                      _fold(ep["att2"]["mlp"]), _fold(ep["mlp2"]),
                      _fold(ep["shortcut"]), bns[i])
        if i == 0:
            fe0 = fe
        fe_prev = fe

    pooled = _sc_gather(fe_prev, sidxs[3]).reshape(NS[4], K, 512)
    xd = _stage_a4(pooled, _fold(p["decoder_0"]), NS[4])

    dec_bns = [704, 704, 512, 512]
    xcur = xd
    tbls = [skips[2], skips[1], skips[0]]
    for j in range(3):
        n = NS[3 - j]
        ii = iidxs[3 - j]
        if ii.shape[0] % 256:
            ii = jnp.pad(ii, (0, 256 - ii.shape[0] % 256))
        fi = _sc_gather(xcur, ii)
        xcur = _stage_dec(n, fi, tbls[j], _fold(p["dec"][j]), dec_bns[j])
    fi = _sc_gather(xcur, iidxs[0])
    out = _stage_head(fi, fe0, _fold(p["dec"][3]), _fold(p["fc1"]),
                      _fold(p["fc2"]),
                      (p["fc"]["W"], p["fc"]["b"]), 512)
    return jnp.transpose(out[None], (0, 2, 1))


# speedup vs baseline: 4.0855x; 3.3516x over previous
import functools

import jax
import jax.numpy as jnp
from jax import lax
from jax.experimental import pallas as pl
from jax.experimental.pallas import tpu as pltpu
from jax.experimental.pallas import tpu_sc as plsc

NS = [45056, 11264, 2816, 704, 176]
K = 16
D_OUT = [16, 64, 128, 256]
D2 = [d // 2 for d in D_OUT]
D_IN = [8, 32, 128, 256]


def _pad16(c):
    return ((c + 15) // 16) * 16


def _leaky(y):
    return jnp.where(y >= 0, y, 0.2 * y)


def _fold(p, pad_out=0):
    w = p["W"] * p["g"][None, :]
    b = p["b"] * p["g"] + p["beta"]
    if pad_out:
        w = jnp.pad(w, ((0, 0), (0, pad_out)))
        b = jnp.pad(b, (0, pad_out))
    return w, b


def _wspec():
    return pl.BlockSpec(memory_space=pl.ANY)


def _full(shape):
    nd = len(shape)
    return pl.BlockSpec(shape, lambda n, _nd=nd: (0,) * _nd)


def _blk(bn, *rest):
    shape = (bn,) + rest
    nd = len(shape)
    return pl.BlockSpec(shape, lambda n, _nd=nd: (n,) + (0,) * (_nd - 1))


_SC_NW = 32


@functools.lru_cache(maxsize=None)
def _make_sc_gather(d, b):
    assert b % (8 * _SC_NW) == 0 and d % 16 == 0
    rows_w = b // _SC_NW
    t = min(128, 32768 // d, rows_w)
    chunks = []
    o = 0
    while o < rows_w:
        chunks.append((o, min(t, rows_w - o)))
        o += t
    m = len(chunks)
    mesh = plsc.VectorSubcoreMesh(core_axis_name="c", subcore_axis_name="s")

    @functools.partial(
        pl.kernel, mesh=mesh,
        out_type=jax.ShapeDtypeStruct((b, d), jnp.float32),
        compiler_params=pltpu.CompilerParams(use_tc_tiling_on_sc=False),
        scratch_types=[
            pltpu.VMEM((rows_w,), jnp.int32),
            pltpu.VMEM((t, d), jnp.float32),
            pltpu.VMEM((t, d), jnp.float32),
            pltpu.SemaphoreType.DMA,
            pltpu.SemaphoreType.DMA,
        ],
    )
    def g(table_hbm, idx_hbm, out_hbm, idx_v, buf0, buf1, sem0, sem1):
        wid = lax.axis_index("s") * 2 + lax.axis_index("c")
        base = wid * rows_w
        pltpu.sync_copy(idx_hbm.at[pl.ds(base, rows_w)], idx_v)
        bufs = (buf0, buf1)
        sems = (sem0, sem1)

        def copy(off, size, p):
            return pltpu.make_async_copy(
                table_hbm.at[idx_v.at[pl.ds(off, size)]],
                bufs[p].at[pl.ds(0, size)], sems[p])

        def finish(off, size, p):
            copy(off, size, p).wait()
            pltpu.sync_copy(bufs[p].at[pl.ds(0, size)],
                            out_hbm.at[pl.ds(base + off, size)])

        if m <= 12:
            copy(chunks[0][0], chunks[0][1], 0).start()
            for ci, (off, sz) in enumerate(chunks):
                if ci + 1 < m:
                    copy(chunks[ci + 1][0], chunks[ci + 1][1],
                         (ci + 1) % 2).start()
                finish(off, sz, ci % 2)
        else:
            assert m % 2 == 0 and all(c[1] == t for c in chunks)
            copy(0, t, 0).start()

            def body(j, carry):
                o0 = 2 * j * t
                copy(o0 + t, t, 1).start()
                finish(o0, t, 0)

                @pl.when(2 * j + 2 < m)
                def _():
                    copy(o0 + 2 * t, t, 0).start()

                finish(o0 + t, t, 1)
                return carry

            lax.fori_loop(0, m // 2, body, 0)

    return g


def _sc_gather(table, idx):
    return _make_sc_gather(table.shape[1], idx.shape[0])(table, idx)


def _stage_a0(features, xyz, fc0, m1, bn):
    n = NS[0]
    d2 = D2[0]
    dg = _pad16(d2 + 3)

    def body(feat_ref, xyz_ref, fw_ref, fb_ref, mw_ref, mb_ref, t_ref, x_ref):
        x = _leaky(feat_ref[...] @ fw_ref[...] + fb_ref[...])
        f = _leaky(x @ mw_ref[...] + mb_ref[...])
        pad = jnp.zeros((bn, dg - d2 - 3), jnp.float32)
        t_ref[...] = jnp.concatenate([f, xyz_ref[...], pad], axis=-1)
        x_ref[...] = x

    fw, fb = fc0
    mw, mb = m1
    return pl.pallas_call(
        body,
        grid=(n // bn,),
        in_specs=[_blk(bn, 3), _blk(bn, 3), _full(fw.shape), _full(fb.shape),
                  _full(mw.shape), _full(mb.shape)],
        out_specs=[_blk(bn, dg), _blk(bn, 8)],
        out_shape=[jax.ShapeDtypeStruct((n, dg), jnp.float32),
                   jax.ShapeDtypeStruct((n, 8), jnp.float32)],
    )(features, xyz, fw, fb, mw, mb)


def _stage_a(i, pooled, xyz, m1, bn):
    n = NS[i]
    dfi = D_IN[i]
    d2 = D2[i]
    dg = _pad16(d2 + 3)

    def body(p_ref, xyz_ref, mw_ref, mb_ref, t_ref, x_ref):
        x = jnp.max(p_ref[...], axis=1)
        f = _leaky(x @ mw_ref[...] + mb_ref[...])
        pad = jnp.zeros((bn, dg - d2 - 3), jnp.float32)
        t_ref[...] = jnp.concatenate([f, xyz_ref[...], pad], axis=-1)
        x_ref[...] = x

    mw, mb = m1
    return pl.pallas_call(
        body,
        grid=(n // bn,),
        in_specs=[_blk(bn, K, dfi), _blk(bn, 3), _full(mw.shape), _full(mb.shape)],
        out_specs=[_blk(bn, dg), _blk(bn, dfi)],
        out_shape=[jax.ShapeDtypeStruct((n, dg), jnp.float32),
                   jax.ShapeDtypeStruct((n, dfi), jnp.float32)],
    )(pooled, xyz, mw, mb)


def _stage_a4(pooled, d0, bn):
    n = NS[4]
    c = 512

    def body(p_ref, w_ref, b_ref, o_ref):
        x = jnp.max(p_ref[...], axis=1)
        o_ref[...] = _leaky(x @ w_ref[...] + b_ref[...])

    w, b = d0
    return pl.pallas_call(
        body,
        grid=(n // bn,),
        in_specs=[_blk(bn, K, c), _full(w.shape), _full(b.shape)],
        out_specs=[_blk(bn, c)],
        out_shape=[jax.ShapeDtypeStruct((n, c), jnp.float32)],
    )(pooled, w, b)[0]


def _stage_d(i, g1, xyz3, xw1, attw, am, xw2, bn):
    n = NS[i]
    d2 = D2[i]
    dg = _pad16(d2 + 3)
    d2p = _pad16(d2)

    def body(g_ref, xyz_ref, x1w_ref, x1b_ref, aw_ref, amw_ref, amb_ref,
             x2w_ref, x2b_ref, fx2_ref, fagg_ref):
        g1b = g_ref[...]
        fnb = g1b[:, :, 0:d2]
        neigh = g1b[:, :, d2:d2 + 3]
        tile = jnp.broadcast_to(xyz_ref[...], (bn, K, 3))
        rel = tile - neigh
        dist = jnp.sqrt(jnp.sum(rel * rel, axis=-1, keepdims=True) + 1e-12)
        relf = jnp.concatenate([dist, rel, tile, neigh], axis=-1)
        relf2 = relf.reshape(bn * K, 10)
        fxyz = _leaky(relf2 @ x1w_ref[...] + x1b_ref[...])
        fset = jnp.concatenate([fnb.reshape(bn * K, d2), fxyz], axis=-1)
        logits = (fset @ aw_ref[...]).reshape(bn, K, 2 * d2)
        m = jnp.max(logits, axis=1, keepdims=True)
        e = jnp.exp(logits - m)
        sc = e / jnp.sum(e, axis=1, keepdims=True)
        agg = jnp.sum(fset.reshape(bn, K, 2 * d2) * sc, axis=1)
        fagg_ref[...] = _leaky(agg @ amw_ref[...] + amb_ref[...])
        fx2_ref[...] = _leaky(fxyz @ x2w_ref[...] + x2b_ref[...]).reshape(bn, K, d2)

    x1w, x1b = xw1
    amw, amb = am
    x2w, x2b = xw2
    return pl.pallas_call(
        body,
        grid=(n // bn,),
        in_specs=[_blk(bn, K, dg), _blk(bn, 1, 3), _full(x1w.shape),
                  _full(x1b.shape), _full(attw.shape), _full(amw.shape),
                  _full(amb.shape), _full(x2w.shape), _full(x2b.shape)],
        out_specs=[_blk(bn, K, d2), _blk(bn, d2p)],
        out_shape=[jax.ShapeDtypeStruct((n, K, d2), jnp.float32),
                   jax.ShapeDtypeStruct((n, d2p), jnp.float32)],
    )(g1, xyz3, x1w, x1b, attw, amw, amb, x2w, x2b)


def _stage_f(i, g2, fxyz2, x, attw, am2, m2, shc, bn):
    n = NS[i]
    d2 = D2[i]
    d2p = _pad16(d2)
    dout = D_OUT[i]
    dfi = D_IN[i]

    def body(g_ref, fx_ref, x_ref, aw_ref, amw_ref, amb_ref, m2w_ref,
             m2b_ref, sw_ref, sb_ref, fe_ref):
        fset = jnp.concatenate(
            [g_ref[...][:, :, 0:d2], fx_ref[...]], axis=-1).reshape(bn * K, 2 * d2)
        logits = (fset @ aw_ref[...]).reshape(bn, K, 2 * d2)
        m = jnp.max(logits, axis=1, keepdims=True)
        e = jnp.exp(logits - m)
        sc = e / jnp.sum(e, axis=1, keepdims=True)
        agg = jnp.sum(fset.reshape(bn, K, 2 * d2) * sc, axis=1)
        a = _leaky(agg @ amw_ref[...] + amb_ref[...])
        f = a @ m2w_ref[...] + m2b_ref[...]
        s = x_ref[...] @ sw_ref[...] + sb_ref[...]
        fe_ref[...] = _leaky(f + s)

    amw, amb = am2
    m2w, m2b = m2
    sw, sb = shc
    return pl.pallas_call(
        body,
        grid=(n // bn,),
        in_specs=[_blk(bn, K, d2p), _blk(bn, K, d2), _blk(bn, dfi),
                  _full(attw.shape), _full(amw.shape), _full(amb.shape),
                  _full(m2w.shape), _full(m2b.shape), _full(sw.shape),
                  _full(sb.shape)],
        out_specs=[_blk(bn, 2 * dout)],
        out_shape=[jax.ShapeDtypeStruct((n, 2 * dout), jnp.float32)],
    )(g2, fxyz2, x, attw, amw, amb, m2w, m2b, sw, sb)[0]


def _stage_dec(n, fi, skip, w_b, bn):
    cs = skip.shape[1]
    ct = fi.shape[1]
    w, b = w_b

    def body(fi_ref, s_ref, w_ref, b_ref, o_ref):
        cat = jnp.concatenate([s_ref[...], fi_ref[...]], axis=-1)
        o_ref[...] = _leaky(cat @ w_ref[...] + b_ref[...])

    return pl.pallas_call(
        body,
        grid=(n // bn,),
        in_specs=[_blk(bn, ct), _blk(bn, cs), _full(w.shape), _full(b.shape)],
        out_specs=[_blk(bn, w.shape[1])],
        out_shape=[jax.ShapeDtypeStruct((n, w.shape[1]), jnp.float32)],
    )(fi[:n], skip, w, b)[0]


def _stage_head(fi, skip, dec3, fc1, fc2, fc, bn):
    n = NS[0]

    def body(fi_ref, s_ref, dw_ref, db_ref, w1_ref, b1_ref, w2_ref, b2_ref,
             w3_ref, b3_ref, o_ref):
        cat = jnp.concatenate([s_ref[...], fi_ref[...]], axis=-1)
        x = _leaky(cat @ dw_ref[...] + db_ref[...])
        x = _leaky(x @ w1_ref[...] + b1_ref[...])
        x = _leaky(x @ w2_ref[...] + b2_ref[...])
        o_ref[...] = x @ w3_ref[...] + b3_ref[...]

    dw, db = dec3
    w1, b1 = fc1
    w2, b2 = fc2
    w3, b3 = fc
    return pl.pallas_call(
        body,
        grid=(n // bn,),
        in_specs=[_blk(bn, 32), _blk(bn, 32)] + [
            _full(a.shape) for a in (dw, db, w1, b1, w2, b2, w3, b3)],
        out_specs=[_blk(bn, 19)],
        out_shape=[jax.ShapeDtypeStruct((n, 19), jnp.float32)],
    )(fi, skip, dw, db, w1, b1, w2, b2, w3, b3)[0]


def kernel(features, xyz_0, xyz_1, xyz_2, xyz_3, neigh_idx_0, neigh_idx_1,
           neigh_idx_2, neigh_idx_3, sub_idx_0, sub_idx_1, sub_idx_2,
           sub_idx_3, interp_idx_0, interp_idx_1, interp_idx_2, interp_idx_3,
           params):
    xyzs = [xyz_0[0], xyz_1[0], xyz_2[0], xyz_3[0]]
    nidxs = [neigh_idx_0[0].reshape(-1), neigh_idx_1[0].reshape(-1),
             neigh_idx_2[0].reshape(-1), neigh_idx_3[0].reshape(-1)]
    sidxs = [sub_idx_0[0].reshape(-1), sub_idx_1[0].reshape(-1),
             sub_idx_2[0].reshape(-1), sub_idx_3[0].reshape(-1)]
    iidxs = [interp_idx_0[0].reshape(-1), interp_idx_1[0].reshape(-1),
             interp_idx_2[0].reshape(-1), interp_idx_3[0].reshape(-1)]

    p = params
    fc0w = p["fc0"]["W"] * p["bn0"]["g"][None, :]
    fc0b = p["fc0"]["b"] * p["bn0"]["g"] + p["bn0"]["beta"]

    bns = [512, 512, 704, 352]
    fe0 = None
    x = None
    skips = []
    t = None
    for i in range(4):
        ep = p["enc"][i]
        d2 = D2[i]
        if i == 0:
            t, x = _stage_a0(features[0], xyzs[0], (fc0w, fc0b),
                             _fold(ep["mlp1"]), bns[0])
        else:
            pooled = _sc_gather(fe_prev, sidxs[i - 1]).reshape(
                NS[i], K, 2 * D_OUT[i - 1])
            t, x = _stage_a(i, pooled, xyzs[i], _fold(ep["mlp1"]), bns[i])
            skips.append(x)
        g1 = _sc_gather(t, nidxs[i]).reshape(NS[i], K, _pad16(d2 + 3))
        fxyz2, fagg = _stage_d(
            i, g1, xyzs[i].reshape(NS[i], 1, 3), _fold(ep["mlp_xyz1"]),
            ep["att1"]["attW"], _fold(ep["att1"]["mlp"], _pad16(d2) - d2),
            _fold(ep["mlp_xyz2"]), bns[i])
        g2 = _sc_gather(fagg, nidxs[i]).reshape(NS[i], K, _pad16(d2))
        fe = _stage_f(i, g2, fxyz2, x, ep["att2"]["attW"],
                      _fold(ep["att2"]["mlp"]), _fold(ep["mlp2"]),
                      _fold(ep["shortcut"]), bns[i])
        if i == 0:
            fe0 = fe
        fe_prev = fe

    pooled = _sc_gather(fe_prev, sidxs[3]).reshape(NS[4], K, 512)
    xd = _stage_a4(pooled, _fold(p["decoder_0"]), NS[4])

    dec_bns = [704, 704, 512, 512]
    xcur = xd
    tbls = [skips[2], skips[1], skips[0]]
    for j in range(3):
        n = NS[3 - j]
        ii = iidxs[3 - j]
        if ii.shape[0] % 256:
            ii = jnp.pad(ii, (0, 256 - ii.shape[0] % 256))
        fi = _sc_gather(xcur, ii)
        xcur = _stage_dec(n, fi, tbls[j], _fold(p["dec"][j]), dec_bns[j])
    fi = _sc_gather(xcur, iidxs[0])
    out = _stage_head(fi, fe0, _fold(p["dec"][3]), _fold(p["fc1"]),
                      _fold(p["fc2"]),
                      (p["fc"]["W"], p["fc"]["b"]), 512)
    return jnp.transpose(out[None], (0, 2, 1))
